# Initial kernel scaffold; baseline (speedup 1.0000x reference)
import jax
import jax.numpy as jnp
from jax.experimental import pallas as pl


def kernel(x, conv1_w, conv1_b, conv2_w, conv2_b, conv3_w, conv3_b, conv4_wx, conv4_wy, conv4_b, t0_ln1_g, t0_ln1_b, t0_qkv_w, t0_out_w, t0_out_b, t0_ln2_g, t0_ln2_b, t0_w1, t0_b1, t0_w2, t0_b2, t1_ln1_g, t1_ln1_b, t1_qkv_w, t1_out_w, t1_out_b, t1_ln2_g, t1_ln2_b, t1_w1, t1_b1, t1_w2, t1_b2):
    raise NotImplementedError("write your pallas kernel here")


# speedup vs baseline: 1.4901x; 1.4901x over previous
import functools

import jax
import jax.numpy as jnp
from jax.experimental import pallas as pl
from jax.experimental.pallas import tpu as pltpu

_VMEM_LIMIT = 32 * 1024 * 1024


def _ru(a, b):
    return ((a + b - 1) // b) * b


def _silu(x):
    return x * (1.0 / (1.0 + jnp.exp(-x)))


def _c12_kernel(w1_ref, b1_ref, w2_ref, b2_ref, x_ref, o_ref, *, Wp, Pout):
    acc = None
    for t in range(9):
        dy, dx = divmod(t, 3)
        st = dy * Wp + dx
        win = x_ref[0, :, st:st + Pout]
        dot = jnp.dot(w1_ref[t], win, preferred_element_type=jnp.float32)
        acc = dot if acc is None else acc + dot
    z = _silu(acc + b1_ref[...])
    y = jnp.dot(w2_ref[...], z, preferred_element_type=jnp.float32)
    o_ref[0] = _silu(y + b2_ref[...]).astype(o_ref.dtype)


def _c4_kernel(wx_ref, wy_ref, b_ref, xa_ref, xb_ref, o_ref, *, Wp, Pout):
    acc = None
    for t in range(9):
        dy, dx = divmod(t, 3)
        st = dy * Wp + dx
        da = jnp.dot(wx_ref[t], xa_ref[0, :, st:st + Pout],
                     preferred_element_type=jnp.float32)
        db = jnp.dot(wy_ref[t], xb_ref[0, :, st:st + Pout],
                     preferred_element_type=jnp.float32)
        d = da + db
        acc = d if acc is None else acc + d
    o_ref[0] = _silu(acc + b_ref[...]).astype(o_ref.dtype)


def _tlayer(x, wqkv, wo, w1, w2, vec, *, n_tok, d, mlp, inner, npatch, eps):
    ln1g, ln1b = vec[0:1, :d], vec[1:2, :d]
    outb = vec[2:3, :d]
    ln2g, ln2b = vec[3:4, :d], vec[4:5, :d]
    b2 = vec[5:6, :d]
    b1 = vec[6:7, :mlp]

    mu = jnp.mean(x, axis=-1, keepdims=True)
    var = jnp.mean(jnp.square(x - mu), axis=-1, keepdims=True)
    xn = (x - mu) * jax.lax.rsqrt(var + eps) * ln1g + ln1b

    qkv = jnp.dot(xn, wqkv, preferred_element_type=jnp.float32)
    avs = []
    for p in range(npatch):
        sl = slice(p * n_tok, (p + 1) * n_tok)
        q = qkv[sl, 0:inner]
        k = qkv[sl, inner:2 * inner]
        v = qkv[sl, 2 * inner:3 * inner]
        s = jax.lax.dot_general(q, k, (((1,), (1,)), ((), ())),
                                preferred_element_type=jnp.float32)
        s = s - jnp.max(s, axis=-1, keepdims=True)
        e = jnp.exp(s)
        a = e / jnp.sum(e, axis=-1, keepdims=True)
        avs.append(jnp.dot(a, v, preferred_element_type=jnp.float32))
    av = jnp.concatenate(avs, axis=0)
    x = x + jnp.dot(av, wo, preferred_element_type=jnp.float32) + outb

    mu2 = jnp.mean(x, axis=-1, keepdims=True)
    var2 = jnp.mean(jnp.square(x - mu2), axis=-1, keepdims=True)
    xn2 = (x - mu2) * jax.lax.rsqrt(var2 + eps) * ln2g + ln2b
    h = _silu(jnp.dot(xn2, w1, preferred_element_type=jnp.float32) + b1)
    return x + jnp.dot(h, w2, preferred_element_type=jnp.float32) + b2


def _vit_kernel(x_ref, wqkv0, wo0, w10, w20, vec0, wqkv1, wo1, w11, w21, vec1,
                w3_ref, b3_ref, o_ref, *, n_tok, d, mlp, inner, npatch, eps):
    x = x_ref[...].astype(jnp.float32)
    kw = dict(n_tok=n_tok, d=d, mlp=mlp, inner=inner, npatch=npatch, eps=eps)
    x = _tlayer(x, wqkv0[...], wo0[...], w10[...], w20[...], vec0[...], **kw)
    x = _tlayer(x, wqkv1[...], wo1[...], w11[...], w21[...], vec1[...], **kw)
    y = jnp.dot(x, w3_ref[...], preferred_element_type=jnp.float32)
    o_ref[...] = _silu(y + b3_ref[...]).astype(o_ref.dtype)


def _pack_vecs(lp, d, mlp):
    maxw = max(d, mlp)

    def row(v):
        v = v.reshape(1, -1)
        return jnp.pad(v, ((0, 0), (0, maxw - v.shape[1])))

    return jnp.concatenate(
        [row(lp["ln1_g"]), row(lp["ln1_b"]), row(lp["out_b"]),
         row(lp["ln2_g"]), row(lp["ln2_b"]), row(lp["b2"]), row(lp["b1"])],
        axis=0)


def kernel(x, conv1_w, conv1_b, conv2_w, conv2_b, conv3_w, conv3_b,
           conv4_wx, conv4_wy, conv4_b,
           t0_ln1_g, t0_ln1_b, t0_qkv_w, t0_out_w, t0_out_b,
           t0_ln2_g, t0_ln2_b, t0_w1, t0_b1, t0_w2, t0_b2,
           t1_ln1_g, t1_ln1_b, t1_qkv_w, t1_out_w, t1_out_b,
           t1_ln2_g, t1_ln2_b, t1_w1, t1_b1, t1_w2, t1_b2):
    N, C, H, W = x.shape
    d = conv2_w.shape[0]
    mlp = t0_w1.shape[1]
    inner = t0_qkv_w.shape[1] // 3
    scale = 32.0 ** -0.5
    ph = pw = 2
    h, w = H // ph, W // pw
    n_tok = h * w
    npatch = ph * pw
    M = npatch * n_tok

    Wp, Hp = W + 2, H + 2
    Pout = _ru(H * Wp, 128)
    Lin = Pout + 2 * Wp + 2

    def padflat(t):
        Ct = t.shape[1]
        tp = jnp.pad(t, ((0, 0), (0, 0), (1, 1), (1, 1)))
        tf = tp.reshape(N, Ct, Hp * Wp)
        return jnp.pad(tf, ((0, 0), (0, 0), (1, Lin - 1 - Hp * Wp)))

    xpad = padflat(x)

    c12 = functools.partial(_c12_kernel, Wp=Wp, Pout=Pout)
    c2 = pl.pallas_call(
        c12,
        out_shape=jax.ShapeDtypeStruct((N, d, Pout), x.dtype),
        grid=(N,),
        in_specs=[
            pl.BlockSpec(conv1_w.shape, lambda n: (0, 0, 0)),
            pl.BlockSpec(conv1_b.shape, lambda n: (0, 0)),
            pl.BlockSpec(conv2_w.shape, lambda n: (0, 0)),
            pl.BlockSpec(conv2_b.shape, lambda n: (0, 0)),
            pl.BlockSpec((1, C, Lin), lambda n: (n, 0, 0)),
        ],
        out_specs=pl.BlockSpec((1, d, Pout), lambda n: (n, 0, 0)),
        compiler_params=pltpu.CompilerParams(
            dimension_semantics=("parallel",),
            vmem_limit_bytes=_VMEM_LIMIT),
    )(conv1_w, conv1_b, conv2_w, conv2_b, xpad)

    t = c2[:, :, :H * Wp].reshape(N, d, H, Wp)[:, :, :, 1:W + 1]
    t = t.reshape(N, d, h, ph, w, pw)
    tok = jnp.transpose(t, (0, 3, 5, 2, 4, 1)).reshape(N * M, d)

    wqkv0 = jnp.concatenate(
        [t0_qkv_w[:, :inner] * scale, t0_qkv_w[:, inner:]], axis=1)
    wqkv1 = jnp.concatenate(
        [t1_qkv_w[:, :inner] * scale, t1_qkv_w[:, inner:]], axis=1)
    vec0 = _pack_vecs({"ln1_g": t0_ln1_g, "ln1_b": t0_ln1_b, "out_b": t0_out_b,
                       "ln2_g": t0_ln2_g, "ln2_b": t0_ln2_b, "b2": t0_b2,
                       "b1": t0_b1}, d, mlp)
    vec1 = _pack_vecs({"ln1_g": t1_ln1_g, "ln1_b": t1_ln1_b, "out_b": t1_out_b,
                       "ln2_g": t1_ln2_g, "ln2_b": t1_ln2_b, "b2": t1_b2,
                       "b1": t1_b1}, d, mlp)
    w3t = conv3_w.T
    b3row = conv3_b.reshape(1, -1)
    maxw = vec0.shape[1]

    const2 = lambda g: (0, 0)
    vit = functools.partial(_vit_kernel, n_tok=n_tok, d=d, mlp=mlp,
                            inner=inner, npatch=npatch, eps=1e-5)
    tok3 = pl.pallas_call(
        vit,
        out_shape=jax.ShapeDtypeStruct((N * M, C), x.dtype),
        grid=(N,),
        in_specs=[
            pl.BlockSpec((M, d), lambda g: (g, 0)),
            pl.BlockSpec((d, 3 * inner), const2),
            pl.BlockSpec((inner, d), const2),
            pl.BlockSpec((d, mlp), const2),
            pl.BlockSpec((mlp, d), const2),
            pl.BlockSpec((7, maxw), const2),
            pl.BlockSpec((d, 3 * inner), const2),
            pl.BlockSpec((inner, d), const2),
            pl.BlockSpec((d, mlp), const2),
            pl.BlockSpec((mlp, d), const2),
            pl.BlockSpec((7, maxw), const2),
            pl.BlockSpec((d, C), const2),
            pl.BlockSpec((1, C), const2),
        ],
        out_specs=pl.BlockSpec((M, C), lambda g: (g, 0)),
        compiler_params=pltpu.CompilerParams(
            dimension_semantics=("parallel",),
            vmem_limit_bytes=_VMEM_LIMIT),
    )(tok, wqkv0, t0_out_w, t0_w1, t0_w2, vec0,
      wqkv1, t1_out_w, t1_w1, t1_w2, vec1, w3t, b3row)

    u = tok3.reshape(N, ph, pw, h, w, C)
    u = jnp.transpose(u, (0, 5, 3, 1, 4, 2)).reshape(N, C, H, W)
    upad = padflat(u)

    c4 = functools.partial(_c4_kernel, Wp=Wp, Pout=Pout)
    o = pl.pallas_call(
        c4,
        out_shape=jax.ShapeDtypeStruct((N, C, Pout), x.dtype),
        grid=(N,),
        in_specs=[
            pl.BlockSpec(conv4_wx.shape, lambda n: (0, 0, 0)),
            pl.BlockSpec(conv4_wy.shape, lambda n: (0, 0, 0)),
            pl.BlockSpec(conv4_b.shape, lambda n: (0, 0)),
            pl.BlockSpec((1, C, Lin), lambda n: (n, 0, 0)),
            pl.BlockSpec((1, C, Lin), lambda n: (n, 0, 0)),
        ],
        out_specs=pl.BlockSpec((1, C, Pout), lambda n: (n, 0, 0)),
        compiler_params=pltpu.CompilerParams(
            dimension_semantics=("parallel",),
            vmem_limit_bytes=_VMEM_LIMIT),
    )(conv4_wx, conv4_wy, conv4_b, upad, xpad)

    return o[:, :, :H * Wp].reshape(N, C, H, Wp)[:, :, :, 1:W + 1]


# speedup vs baseline: 3.5228x; 2.3642x over previous
import functools

import jax
import jax.numpy as jnp
from jax.experimental import pallas as pl
from jax.experimental.pallas import tpu as pltpu

_VMEM_LIMIT = 32 * 1024 * 1024
_MARGIN = 40


def _silu(x):
    return (0.5 * x) * (1.0 + jnp.tanh(0.5 * x))


def _conv3x3_tokmajor(scr_ref, wt_ref, P, W, nC):
    spans = {e: scr_ref[_MARGIN - W + e:_MARGIN + W + e + P, :]
             for e in (-1, 0, 1)}
    accs = {}
    for t in range(9):
        dy, dx = divmod(t, 3)
        win = spans[dx - 1][W * dy:W * dy + P, :]
        dot = jnp.dot(win, wt_ref[t], preferred_element_type=jnp.float32)
        accs[dx] = dot if dx not in accs else accs[dx] + dot
    rows = jax.lax.broadcasted_iota(jnp.int32, accs[0].shape, 0) % W
    out = accs[1]
    out = out + jnp.where(rows != 0, accs[0], 0.0)
    return out + jnp.where(rows != W - 1, accs[2], 0.0)


def _c12_kernel(w1_ref, b1_ref, w2_ref, b2_ref, x_ref, o_ref, scr_ref,
                *, P, W, C):
    scr_ref[0:_MARGIN, :] = jnp.zeros((_MARGIN, C), scr_ref.dtype)
    scr_ref[_MARGIN + P:, :] = jnp.zeros(
        (scr_ref.shape[0] - _MARGIN - P, C), scr_ref.dtype)
    scr_ref[_MARGIN:_MARGIN + P, :] = x_ref[0]
    acc = _conv3x3_tokmajor(scr_ref, w1_ref, P, W, C)
    z1 = _silu(acc + b1_ref[...])
    z2 = jnp.dot(z1, w2_ref[...], preferred_element_type=jnp.float32)
    z2 = _silu(z2 + b2_ref[...])
    o_ref[0] = _patchify_rows(z2, W // 2, W // 2).astype(o_ref.dtype)


def _c4_kernel(w4_ref, b4_ref, z3_ref, x_ref, o_ref, scr_ref, *, P, W, C):
    scr_ref[0:_MARGIN, :] = jnp.zeros((_MARGIN, 2 * C), scr_ref.dtype)
    scr_ref[_MARGIN + P:, :] = jnp.zeros(
        (scr_ref.shape[0] - _MARGIN - P, 2 * C), scr_ref.dtype)
    scr_ref[_MARGIN:_MARGIN + P, 0:C] = _unpatchify_rows(
        z3_ref[0].astype(jnp.float32), W // 2, W // 2)
    scr_ref[_MARGIN:_MARGIN + P, C:2 * C] = x_ref[0]
    acc = _conv3x3_tokmajor(scr_ref, w4_ref, P, W, 2 * C)
    o_ref[0] = _silu(acc + b4_ref[...]).astype(o_ref.dtype)


def _patchify_rows(a, h, w):
    C = a.shape[1]
    a = a.reshape(h, 2, w, 2, C)
    return jnp.transpose(a, (1, 3, 0, 2, 4)).reshape(4 * h * w, C)


def _unpatchify_rows(a, h, w):
    C = a.shape[1]
    a = a.reshape(2, 2, h, w, C)
    return jnp.transpose(a, (2, 0, 3, 1, 4)).reshape(4 * h * w, C)


def _layernorm(x, lnones, g, b, d, eps):
    cat = jnp.concatenate([x, x * x], axis=1)
    sums = jnp.dot(cat, lnones, preferred_element_type=jnp.float32)
    mu = sums[:, 0:d]
    var = sums[:, d:2 * d] - mu * mu
    return (x - mu) * jax.lax.rsqrt(var + eps) * g + b


def _tlayer(x, wqkv, wo, w1, w2, vec, lnones, *, n_tok, d, mlp, inner,
            npatch, eps):
    ln1g, ln1b = vec[0:1, :d], vec[1:2, :d]
    outb = vec[2:3, :d]
    ln2g, ln2b = vec[3:4, :d], vec[4:5, :d]
    b2 = vec[5:6, :d]
    b1 = vec[6:7, :mlp]

    xn = _layernorm(x, lnones, ln1g, ln1b, d, eps)
    qkv = jnp.dot(xn, wqkv, preferred_element_type=jnp.float32)
    avs = []
    for p in range(npatch):
        sl = slice(p * n_tok, (p + 1) * n_tok)
        q = qkv[sl, 0:inner]
        k = qkv[sl, inner:2 * inner]
        v = qkv[sl, 2 * inner:3 * inner]
        s = jax.lax.dot_general(q, k, (((1,), (1,)), ((), ())),
                                preferred_element_type=jnp.float32)
        e = jnp.exp(s - jnp.max(s, axis=-1, keepdims=True))
        a = e / jnp.sum(e, axis=-1, keepdims=True)
        avs.append(jnp.dot(a, v, preferred_element_type=jnp.float32))
    av = jnp.concatenate(avs, axis=0)
    x = x + jnp.dot(av, wo, preferred_element_type=jnp.float32) + outb

    xn2 = _layernorm(x, lnones, ln2g, ln2b, d, eps)
    hh = _silu(jnp.dot(xn2, w1, preferred_element_type=jnp.float32) + b1)
    return x + jnp.dot(hh, w2, preferred_element_type=jnp.float32) + b2


def _vit_kernel(x_ref, wqkv0, wo0, w10, w20, vec0, wqkv1, wo1, w11, w21, vec1,
                w3_ref, b3_ref, ln_ref, o_ref, *, n_tok, d, mlp, inner,
                npatch, h, w, eps):
    nb, P = x_ref.shape[0], x_ref.shape[1]
    x = x_ref[...].astype(jnp.float32).reshape(nb * P, d)
    lnones = ln_ref[...]
    kw = dict(n_tok=n_tok, d=d, mlp=mlp, inner=inner, npatch=nb * npatch,
              eps=eps)
    x = _tlayer(x, wqkv0[...], wo0[...], w10[...], w20[...], vec0[...],
                lnones, **kw)
    x = _tlayer(x, wqkv1[...], wo1[...], w11[...], w21[...], vec1[...],
                lnones, **kw)
    y = jnp.dot(x, w3_ref[...], preferred_element_type=jnp.float32)
    y = _silu(y + b3_ref[...])
    o_ref[...] = y.reshape(nb, P, y.shape[1]).astype(o_ref.dtype)


def _pack_vecs(lp, d, mlp):
    maxw = max(d, mlp)

    def row(v):
        v = v.reshape(1, -1)
        return jnp.pad(v, ((0, 0), (0, maxw - v.shape[1])))

    return jnp.concatenate(
        [row(lp["ln1_g"]), row(lp["ln1_b"]), row(lp["out_b"]),
         row(lp["ln2_g"]), row(lp["ln2_b"]), row(lp["b2"]), row(lp["b1"])],
        axis=0)


def kernel(x, conv1_w, conv1_b, conv2_w, conv2_b, conv3_w, conv3_b,
           conv4_wx, conv4_wy, conv4_b,
           t0_ln1_g, t0_ln1_b, t0_qkv_w, t0_out_w, t0_out_b,
           t0_ln2_g, t0_ln2_b, t0_w1, t0_b1, t0_w2, t0_b2,
           t1_ln1_g, t1_ln1_b, t1_qkv_w, t1_out_w, t1_out_b,
           t1_ln2_g, t1_ln2_b, t1_w1, t1_b1, t1_w2, t1_b2):
    N, C, H, W = x.shape
    d = conv2_w.shape[0]
    mlp = t0_w1.shape[1]
    inner = t0_qkv_w.shape[1] // 3
    scale = 32.0 ** -0.5
    h, w = H // 2, W // 2
    n_tok = h * w
    npatch = 4
    P = H * W
    SCR = _MARGIN + P + _MARGIN

    xf = jnp.transpose(x.reshape(N, C, P), (0, 2, 1))

    w1t = jnp.transpose(conv1_w, (0, 2, 1))
    w2t = conv2_w.T
    c12 = functools.partial(_c12_kernel, P=P, W=W, C=C)
    tok = pl.pallas_call(
        c12,
        out_shape=jax.ShapeDtypeStruct((N, P, d), x.dtype),
        grid=(N,),
        in_specs=[
            pl.BlockSpec(w1t.shape, lambda n: (0, 0, 0)),
            pl.BlockSpec((1, C), lambda n: (0, 0)),
            pl.BlockSpec(w2t.shape, lambda n: (0, 0)),
            pl.BlockSpec((1, d), lambda n: (0, 0)),
            pl.BlockSpec((1, P, C), lambda n: (n, 0, 0)),
        ],
        out_specs=pl.BlockSpec((1, P, d), lambda n: (n, 0, 0)),
        scratch_shapes=[pltpu.VMEM((SCR, C), jnp.float32)],
        compiler_params=pltpu.CompilerParams(
            dimension_semantics=("parallel",),
            vmem_limit_bytes=_VMEM_LIMIT),
    )(w1t, conv1_b.reshape(1, C), w2t, conv2_b.reshape(1, d), xf)

    wqkv0 = jnp.concatenate(
        [t0_qkv_w[:, :inner] * scale, t0_qkv_w[:, inner:]], axis=1)
    wqkv1 = jnp.concatenate(
        [t1_qkv_w[:, :inner] * scale, t1_qkv_w[:, inner:]], axis=1)
    vec0 = _pack_vecs({"ln1_g": t0_ln1_g, "ln1_b": t0_ln1_b, "out_b": t0_out_b,
                       "ln2_g": t0_ln2_g, "ln2_b": t0_ln2_b, "b2": t0_b2,
                       "b1": t0_b1}, d, mlp)
    vec1 = _pack_vecs({"ln1_g": t1_ln1_g, "ln1_b": t1_ln1_b, "out_b": t1_out_b,
                       "ln2_g": t1_ln2_g, "ln2_b": t1_ln2_b, "b2": t1_b2,
                       "b1": t1_b1}, d, mlp)
    w3t = conv3_w.T
    b3row = conv3_b.reshape(1, C)
    maxw = vec0.shape[1]
    o_dd = jnp.full((d, d), 1.0 / d, jnp.float32)
    z_dd = jnp.zeros((d, d), jnp.float32)
    lnones = jnp.concatenate(
        [jnp.concatenate([o_dd, z_dd], axis=1),
         jnp.concatenate([z_dd, o_dd], axis=1)], axis=0)

    const2 = lambda g: (0, 0)
    vit = functools.partial(_vit_kernel, n_tok=n_tok, d=d, mlp=mlp,
                            inner=inner, npatch=npatch, h=h, w=w, eps=1e-5)
    NB = 2 if N % 2 == 0 else 1
    z3 = pl.pallas_call(
        vit,
        out_shape=jax.ShapeDtypeStruct((N, P, C), x.dtype),
        grid=(N // NB,),
        in_specs=[
            pl.BlockSpec((NB, P, d), lambda g: (g, 0, 0)),
            pl.BlockSpec((d, 3 * inner), const2),
            pl.BlockSpec((inner, d), const2),
            pl.BlockSpec((d, mlp), const2),
            pl.BlockSpec((mlp, d), const2),
            pl.BlockSpec((7, maxw), const2),
            pl.BlockSpec((d, 3 * inner), const2),
            pl.BlockSpec((inner, d), const2),
            pl.BlockSpec((d, mlp), const2),
            pl.BlockSpec((mlp, d), const2),
            pl.BlockSpec((7, maxw), const2),
            pl.BlockSpec((d, C), const2),
            pl.BlockSpec((1, C), const2),
            pl.BlockSpec((2 * d, 2 * d), const2),
        ],
        out_specs=pl.BlockSpec((NB, P, C), lambda g: (g, 0, 0)),
        compiler_params=pltpu.CompilerParams(
            dimension_semantics=("parallel",),
            vmem_limit_bytes=_VMEM_LIMIT),
    )(tok, wqkv0, t0_out_w, t0_w1, t0_w2, vec0,
      wqkv1, t1_out_w, t1_w1, t1_w2, vec1, w3t, b3row, lnones)

    w4t = jnp.concatenate([jnp.transpose(conv4_wx, (0, 2, 1)),
                           jnp.transpose(conv4_wy, (0, 2, 1))], axis=1)
    c4 = functools.partial(_c4_kernel, P=P, W=W, C=C)
    o = pl.pallas_call(
        c4,
        out_shape=jax.ShapeDtypeStruct((N, P, C), x.dtype),
        grid=(N,),
        in_specs=[
            pl.BlockSpec(w4t.shape, lambda n: (0, 0, 0)),
            pl.BlockSpec((1, C), lambda n: (0, 0)),
            pl.BlockSpec((1, P, C), lambda n: (n, 0, 0)),
            pl.BlockSpec((1, P, C), lambda n: (n, 0, 0)),
        ],
        out_specs=pl.BlockSpec((1, P, C), lambda n: (n, 0, 0)),
        scratch_shapes=[pltpu.VMEM((SCR, 2 * C), jnp.float32)],
        compiler_params=pltpu.CompilerParams(
            dimension_semantics=("parallel",),
            vmem_limit_bytes=_VMEM_LIMIT),
    )(w4t, conv4_b.reshape(1, C), z3, xf)

    return jnp.transpose(o, (0, 2, 1)).reshape(N, C, H, W)
